# baseline (speedup 1.0000x reference)
import jax
import jax.numpy as jnp
from jax import lax
from jax.experimental import pallas as pl
from jax.experimental.pallas import tpu as pltpu

_S = 30.0
_M = 0.35
_LANES = 128


def _stream_body(n_cols, n_blocks, bc, x_ref, lbl_ref, out_ref,
                 m_ref, s_ref, tg_ref):
    i = pl.program_id(0)
    nch = bc // _LANES

    @pl.when(i == 0)
    def _init():
        m_ref[...] = jnp.full_like(m_ref, -jnp.inf)
        s_ref[...] = jnp.zeros_like(s_ref)
        tg_ref[...] = jnp.zeros_like(tg_ref)

    lbl = lbl_ref[...]
    g = lbl // _LANES

    def update(chunks):
        bm = chunks[0]
        for c in chunks[1:]:
            bm = jnp.maximum(bm, c)
        m_old = m_ref[...]
        m_new = jnp.maximum(m_old, bm)
        acc = jnp.exp(_S * (chunks[0] - m_new))
        for c in chunks[1:]:
            acc = acc + jnp.exp(_S * (c - m_new))
        s_ref[...] = s_ref[...] * jnp.exp(_S * (m_old - m_new)) + acc
        m_ref[...] = m_new

    def capture(raw_chunks):
        tg = tg_ref[...]
        for c, chunk in enumerate(raw_chunks):
            tg = jnp.where(g == i * nch + c, chunk, tg)
        tg_ref[...] = tg

    @pl.when(i < n_blocks - 1)
    def _main():
        xb = x_ref[...]
        chunks = [xb[:, c * _LANES:(c + 1) * _LANES] for c in range(nch)]
        update(chunks)
        capture(chunks)

    @pl.when(i == n_blocks - 1)
    def _tail():
        xb = x_ref[...]
        raw = [xb[:, c * _LANES:(c + 1) * _LANES] for c in range(nch)]
        lane = lax.broadcasted_iota(jnp.int32, (xb.shape[0], _LANES), 1)
        masked = [jnp.where(i * bc + c * _LANES + lane < n_cols, rc, -jnp.inf)
                  for c, rc in enumerate(raw)]
        update(masked)
        capture(raw)

        m = m_ref[...]
        mrow = jnp.max(m, axis=1, keepdims=True)
        srow = jnp.sum(s_ref[...] * jnp.exp(_S * (m - mrow)),
                       axis=1, keepdims=True)
        lmatch = lane == lbl % _LANES
        t = jnp.sum(jnp.where(lmatch, tg_ref[...], 0.0), axis=1, keepdims=True)
        e1 = jnp.exp(_S * (t - mrow))
        e2 = jnp.exp(_S * (t - _M - mrow))
        s_corr = jnp.maximum(srow - e1, 0.0) + e2
        nll = jnp.log(s_corr) + _S * mrow - _S * (t - _M)
        out_ref[...] = jnp.sum(nll, axis=(0, 1), keepdims=True) / nll.shape[0]


@jax.jit
def kernel(input, label):
    n_rows, n_cols = input.shape
    lbl = label.astype(jnp.int32).reshape(n_rows, 1)

    bc = 1024
    n_blocks = pl.cdiv(n_cols, bc)
    body = lambda *refs: _stream_body(n_cols, n_blocks, bc, *refs)
    out = pl.pallas_call(
        body,
        grid=(n_blocks,),
        in_specs=[
            pl.BlockSpec((n_rows, bc), lambda i: (0, i)),
            pl.BlockSpec((n_rows, 1), lambda i: (0, 0)),
        ],
        out_specs=pl.BlockSpec((1, 1), lambda i: (0, 0)),
        out_shape=jax.ShapeDtypeStruct((1, 1), jnp.float32),
        scratch_shapes=[
            pltpu.VMEM((n_rows, _LANES), jnp.float32),
            pltpu.VMEM((n_rows, _LANES), jnp.float32),
            pltpu.VMEM((n_rows, _LANES), jnp.float32),
        ],
    )(input, lbl)
    return out[0, 0]


# speedup vs baseline: 1.1885x; 1.1885x over previous
import jax
import jax.numpy as jnp
from jax import lax
from jax.experimental import pallas as pl
from jax.experimental.pallas import tpu as pltpu

_S = 30.0
_M = 0.35
_LANES = 128


def _stream_body(n_cols, n_blocks, bc, x_ref, gb_ref, laneq_ref, out_ref,
                 m_ref, s_ref, tg_ref):
    i = pl.program_id(0)
    nch = bc // _LANES

    @pl.when(i == 0)
    def _init():
        m_ref[...] = jnp.full_like(m_ref, -jnp.inf)
        s_ref[...] = jnp.zeros_like(s_ref)
        tg_ref[...] = jnp.zeros_like(tg_ref)

    def chunk(c):
        return x_ref[:, c * _LANES:(c + 1) * _LANES]

    def update_and_capture(mask_tail):
        def masked(c):
            xc = chunk(c)
            if not mask_tail:
                return xc
            lane = lax.broadcasted_iota(jnp.int32, xc.shape, 1)
            return jnp.where(i * bc + c * _LANES + lane < n_cols, xc, -jnp.inf)

        bm = masked(0)
        for c in range(1, nch):
            bm = jnp.maximum(bm, masked(c))
        m_old = m_ref[...]
        m_new = jnp.maximum(m_old, bm)
        m_ref[...] = m_new

        gb = gb_ref[...]
        tg = tg_ref[...]
        acc = jnp.zeros_like(m_new)
        for c in range(nch):
            acc = acc + jnp.exp(_S * (masked(c) - m_new))
            tg = jnp.where(gb == i * nch + c, chunk(c), tg)
        tg_ref[...] = tg
        s_ref[...] = s_ref[...] * jnp.exp(_S * (m_old - m_new)) + acc

    @pl.when(i < n_blocks - 1)
    def _main():
        update_and_capture(False)

    @pl.when(i == n_blocks - 1)
    def _tail():
        update_and_capture(True)

        m = m_ref[...]
        mrow = jnp.max(m, axis=1, keepdims=True)
        srow = jnp.sum(s_ref[...] * jnp.exp(_S * (m - mrow)),
                       axis=1, keepdims=True)
        t = jnp.sum(jnp.where(laneq_ref[...] != 0.0, tg_ref[...], 0.0),
                    axis=1, keepdims=True)
        e1 = jnp.exp(_S * (t - mrow))
        e2 = jnp.exp(_S * (t - _M - mrow))
        s_corr = jnp.maximum(srow - e1, 0.0) + e2
        nll = jnp.log(s_corr) + _S * mrow - _S * (t - _M)
        out_ref[...] = jnp.sum(nll, axis=(0, 1), keepdims=True) / nll.shape[0]


@jax.jit
def kernel(input, label):
    n_rows, n_cols = input.shape
    lbl = label.astype(jnp.int32)

    gb = jnp.broadcast_to((lbl // _LANES)[:, None], (n_rows, _LANES))
    laneq = (lbl[:, None] % _LANES ==
             jnp.arange(_LANES, dtype=jnp.int32)[None, :]).astype(jnp.float32)

    bc = 1024
    n_blocks = pl.cdiv(n_cols, bc)
    body = lambda *refs: _stream_body(n_cols, n_blocks, bc, *refs)
    out = pl.pallas_call(
        body,
        grid=(n_blocks,),
        in_specs=[
            pl.BlockSpec((n_rows, bc), lambda i: (0, i)),
            pl.BlockSpec((n_rows, _LANES), lambda i: (0, 0)),
            pl.BlockSpec((n_rows, _LANES), lambda i: (0, 0)),
        ],
        out_specs=pl.BlockSpec((1, 1), lambda i: (0, 0)),
        out_shape=jax.ShapeDtypeStruct((1, 1), jnp.float32),
        scratch_shapes=[
            pltpu.VMEM((n_rows, _LANES), jnp.float32),
            pltpu.VMEM((n_rows, _LANES), jnp.float32),
            pltpu.VMEM((n_rows, _LANES), jnp.float32),
        ],
    )(input, gb, laneq)
    return out[0, 0]


# speedup vs baseline: 1.2059x; 1.0147x over previous
import jax
import jax.numpy as jnp
from jax import lax
from jax.experimental import pallas as pl
from jax.experimental.pallas import tpu as pltpu

_S = 30.0
_M = 0.35
_LANES = 128


def _stripe_body(n_rows, n_cols, x_ref, gb_ref, laneq_ref, out_ref):
    i = pl.program_id(0)
    nch = pl.cdiv(n_cols, _LANES)
    rem = n_cols - (nch - 1) * _LANES

    @pl.when(i == 0)
    def _init():
        out_ref[...] = jnp.zeros_like(out_ref)

    def chunk(c):
        xc = x_ref[:, c * _LANES:(c + 1) * _LANES]
        if c == nch - 1 and rem != _LANES:
            lane = lax.broadcasted_iota(jnp.int32, xc.shape, 1)
            xc = jnp.where(lane < rem, xc, -jnp.inf)
        return xc

    gb = gb_ref[...]
    bm = chunk(0)
    tg = jnp.where(gb == 0, x_ref[:, 0:_LANES], bm)
    for c in range(1, nch):
        xc = chunk(c)
        bm = jnp.maximum(bm, xc)
        tg = jnp.where(gb == c, x_ref[:, c * _LANES:(c + 1) * _LANES], tg)

    acc = jnp.exp(_S * (chunk(0) - bm))
    for c in range(1, nch):
        acc = acc + jnp.exp(_S * (chunk(c) - bm))

    mrow = jnp.max(bm, axis=1, keepdims=True)
    srow = jnp.sum(acc * jnp.exp(_S * (bm - mrow)), axis=1, keepdims=True)
    t = jnp.sum(jnp.where(laneq_ref[...] != 0.0, tg, 0.0),
                axis=1, keepdims=True)
    e1 = jnp.exp(_S * (t - mrow))
    e2 = jnp.exp(_S * (t - _M - mrow))
    s_corr = jnp.maximum(srow - e1, 0.0) + e2
    nll = jnp.log(s_corr) + _S * mrow - _S * (t - _M)
    out_ref[...] = out_ref[...] + \
        jnp.sum(nll, axis=(0, 1), keepdims=True) / n_rows


@jax.jit
def kernel(input, label):
    n_rows, n_cols = input.shape
    lbl = label.astype(jnp.int32)

    gb = jnp.broadcast_to((lbl // _LANES)[:, None], (n_rows, _LANES))
    laneq = (lbl[:, None] % _LANES ==
             jnp.arange(_LANES, dtype=jnp.int32)[None, :]).astype(jnp.float32)

    rb = 16
    bc = pl.cdiv(n_cols, _LANES) * _LANES
    body = lambda *refs: _stripe_body(n_rows, n_cols, *refs)
    out = pl.pallas_call(
        body,
        grid=(n_rows // rb,),
        in_specs=[
            pl.BlockSpec((rb, bc), lambda i: (i, 0)),
            pl.BlockSpec((rb, _LANES), lambda i: (i, 0)),
            pl.BlockSpec((rb, _LANES), lambda i: (i, 0)),
        ],
        out_specs=pl.BlockSpec((1, 1), lambda i: (0, 0)),
        out_shape=jax.ShapeDtypeStruct((1, 1), jnp.float32),
    )(input, gb, laneq)
    return out[0, 0]
